# baseline (speedup 1.0000x reference)
import functools

import jax
import jax.numpy as jnp
from jax import lax
from jax.experimental import pallas as pl
from jax.experimental.pallas import tpu as pltpu
from jax.experimental.pallas import tpu_sc as plsc

_NC = 2
_NS = 16
_LANES = 16
_NW = _NC * _NS
_CHUNKS = 2


def _mesh():
    return plsc.VectorSubcoreMesh(core_axis_name="c", subcore_axis_name="s")


@functools.lru_cache(maxsize=None)
def _build_k1(batch: int, n_fields: int):
    s_per_w = batch // _NW
    s_chunk = s_per_w // _CHUNKS
    g_chunk = s_chunk // _LANES
    n_per_w = s_per_w * n_fields

    @functools.partial(
        pl.kernel,
        mesh=_mesh(),
        out_type=jax.ShapeDtypeStruct((batch * n_fields,), jnp.int32),
        compiler_params=pltpu.CompilerParams(needs_layout_passes=False),
        scratch_types=[
            pltpu.VMEM((s_per_w, n_fields), jnp.int32),
            pltpu.VMEM((n_per_w,), jnp.int32),
        ],
    )
    def k1(x_hbm, idx_hbm, x_v, idx_v):
        wid = lax.axis_index("s") * _NC + lax.axis_index("c")
        pltpu.sync_copy(x_hbm.at[pl.ds(wid * s_per_w, s_per_w), :], x_v)
        lane = lax.iota(jnp.int32, _LANES)

        def grp(t, carry):
            c = t // g_chunk
            g = t - c * g_chunk
            sl = c * s_chunk + g * _LANES + lane
            base = c * s_chunk * n_fields + g * _LANES
            for j in range(n_fields):
                v = plsc.load_gather(x_v, [sl, lane * 0 + j])
                idx_v[pl.ds(base + j * s_chunk, _LANES)] = v
            return carry

        lax.fori_loop(0, _CHUNKS * g_chunk, grp, 0)
        pltpu.sync_copy(idx_v, idx_hbm.at[pl.ds(wid * n_per_w, n_per_w)])

    return k1


@functools.lru_cache(maxsize=None)
def _build_k2(batch: int, n_fields: int):
    s_per_w = batch // _NW
    s_chunk = s_per_w // _CHUNKS
    g_chunk = s_chunk // _LANES
    n_per_w = s_per_w * n_fields
    n_chunk = s_chunk * n_fields

    @functools.partial(
        pl.kernel,
        mesh=_mesh(),
        out_type=jax.ShapeDtypeStruct((batch,), jnp.float32),
        compiler_params=pltpu.CompilerParams(needs_layout_passes=False),
        scratch_types=(
            [pltpu.VMEM((n_per_w,), jnp.int32)]
            + [pltpu.VMEM((n_chunk,), jnp.float32) for _ in range(_CHUNKS)]
            + [pltpu.VMEM((_LANES,), jnp.float32),
               pltpu.VMEM((s_per_w,), jnp.float32)]
            + [pltpu.SemaphoreType.DMA for _ in range(_CHUNKS)]
        ),
    )
    def k2(idx_hbm, tab_hbm, bias_hbm, out_hbm, *scr):
        idx_v = scr[0]
        val_vs = scr[1:1 + _CHUNKS]
        bias_v, out_v = scr[1 + _CHUNKS:3 + _CHUNKS]
        sems = scr[3 + _CHUNKS:]
        wid = lax.axis_index("s") * _NC + lax.axis_index("c")
        pltpu.sync_copy(idx_hbm.at[pl.ds(wid * n_per_w, n_per_w)], idx_v)
        pltpu.sync_copy(bias_hbm, bias_v)
        bias_vec = bias_v[...]

        def start(c):
            return pltpu.async_copy(
                tab_hbm.at[idx_v.at[pl.ds(c * n_chunk, n_chunk)]],
                val_vs[c], sems[c])

        def reduce(c):
            val_v = val_vs[c]

            def grp(g, carry):
                acc = bias_vec
                for j in range(n_fields):
                    acc = acc + val_v[pl.ds(j * s_chunk + g * _LANES, _LANES)]
                sig = 1.0 / (1.0 + jnp.exp(-acc))
                out_v[pl.ds(c * s_chunk + g * _LANES, _LANES)] = sig
                return carry

            lax.fori_loop(0, g_chunk, grp, 0)

        dmas = [start(c) for c in range(_CHUNKS)]
        for c in range(_CHUNKS):
            dmas[c].wait()
            reduce(c)
        pltpu.sync_copy(out_v, out_hbm.at[pl.ds(wid * s_per_w, s_per_w)])

    return k2


def kernel(x, table, bias):
    batch, n_fields = x.shape
    tab_flat = table.astype(jnp.float32).reshape(-1)
    bias16 = jnp.broadcast_to(
        bias.reshape(-1)[:1], (_LANES,)).astype(jnp.float32)
    idx_flat = _build_k1(batch, n_fields)(x.astype(jnp.int32))
    out = _build_k2(batch, n_fields)(idx_flat, tab_flat, bias16)
    return out.reshape(batch, 1)


# speedup vs baseline: 1.0070x; 1.0070x over previous
import functools

import jax
import jax.numpy as jnp
from jax import lax
from jax.experimental import pallas as pl
from jax.experimental.pallas import tpu as pltpu
from jax.experimental.pallas import tpu_sc as plsc

_NC = 2
_NS = 16
_LANES = 16
_NW = _NC * _NS
_CHUNKS = 4


def _mesh():
    return plsc.VectorSubcoreMesh(core_axis_name="c", subcore_axis_name="s")


@functools.lru_cache(maxsize=None)
def _build_k1(batch: int, n_fields: int):
    s_per_w = batch // _NW
    s_chunk = s_per_w // _CHUNKS
    g_chunk = s_chunk // _LANES
    n_per_w = s_per_w * n_fields

    @functools.partial(
        pl.kernel,
        mesh=_mesh(),
        out_type=jax.ShapeDtypeStruct((batch * n_fields,), jnp.int32),
        compiler_params=pltpu.CompilerParams(needs_layout_passes=False),
        scratch_types=[
            pltpu.VMEM((s_per_w, n_fields), jnp.int32),
            pltpu.VMEM((n_per_w,), jnp.int32),
        ],
    )
    def k1(x_hbm, idx_hbm, x_v, idx_v):
        wid = lax.axis_index("s") * _NC + lax.axis_index("c")
        pltpu.sync_copy(x_hbm.at[pl.ds(wid * s_per_w, s_per_w), :], x_v)
        lane = lax.iota(jnp.int32, _LANES)

        def grp(t, carry):
            c = t // g_chunk
            g = t - c * g_chunk
            sl = c * s_chunk + g * _LANES + lane
            base = c * s_chunk * n_fields + g * _LANES
            for j in range(n_fields):
                v = plsc.load_gather(x_v, [sl, lane * 0 + j])
                idx_v[pl.ds(base + j * s_chunk, _LANES)] = v
            return carry

        lax.fori_loop(0, _CHUNKS * g_chunk, grp, 0)
        pltpu.sync_copy(idx_v, idx_hbm.at[pl.ds(wid * n_per_w, n_per_w)])

    return k1


@functools.lru_cache(maxsize=None)
def _build_k2(batch: int, n_fields: int):
    s_per_w = batch // _NW
    s_chunk = s_per_w // _CHUNKS
    g_chunk = s_chunk // _LANES
    n_per_w = s_per_w * n_fields
    n_chunk = s_chunk * n_fields

    @functools.partial(
        pl.kernel,
        mesh=_mesh(),
        out_type=jax.ShapeDtypeStruct((batch,), jnp.float32),
        compiler_params=pltpu.CompilerParams(needs_layout_passes=False),
        scratch_types=(
            [pltpu.VMEM((n_per_w,), jnp.int32)]
            + [pltpu.VMEM((n_chunk,), jnp.float32) for _ in range(_CHUNKS)]
            + [pltpu.VMEM((_LANES,), jnp.float32),
               pltpu.VMEM((s_per_w,), jnp.float32)]
            + [pltpu.SemaphoreType.DMA for _ in range(_CHUNKS)]
        ),
    )
    def k2(idx_hbm, tab_hbm, bias_hbm, out_hbm, *scr):
        idx_v = scr[0]
        val_vs = scr[1:1 + _CHUNKS]
        bias_v, out_v = scr[1 + _CHUNKS:3 + _CHUNKS]
        sems = scr[3 + _CHUNKS:]
        wid = lax.axis_index("s") * _NC + lax.axis_index("c")
        pltpu.sync_copy(idx_hbm.at[pl.ds(wid * n_per_w, n_per_w)], idx_v)
        pltpu.sync_copy(bias_hbm, bias_v)
        bias_vec = bias_v[...]

        def start(c):
            return pltpu.async_copy(
                tab_hbm.at[idx_v.at[pl.ds(c * n_chunk, n_chunk)]],
                val_vs[c], sems[c])

        def reduce(c):
            val_v = val_vs[c]

            def grp(g, carry):
                acc = bias_vec
                for j in range(n_fields):
                    acc = acc + val_v[pl.ds(j * s_chunk + g * _LANES, _LANES)]
                sig = 1.0 / (1.0 + jnp.exp(-acc))
                out_v[pl.ds(c * s_chunk + g * _LANES, _LANES)] = sig
                return carry

            lax.fori_loop(0, g_chunk, grp, 0)

        dmas = [start(c) for c in range(_CHUNKS)]
        for c in range(_CHUNKS):
            dmas[c].wait()
            reduce(c)
        pltpu.sync_copy(out_v, out_hbm.at[pl.ds(wid * s_per_w, s_per_w)])

    return k2


def kernel(x, table, bias):
    batch, n_fields = x.shape
    tab_flat = table.astype(jnp.float32).reshape(-1)
    bias16 = jnp.broadcast_to(
        bias.reshape(-1)[:1], (_LANES,)).astype(jnp.float32)
    idx_flat = _build_k1(batch, n_fields)(x.astype(jnp.int32))
    out = _build_k2(batch, n_fields)(idx_flat, tab_flat, bias16)
    return out.reshape(batch, 1)


# speedup vs baseline: 1.0143x; 1.0072x over previous
import functools

import jax
import jax.numpy as jnp
from jax import lax
from jax.experimental import pallas as pl
from jax.experimental.pallas import tpu as pltpu
from jax.experimental.pallas import tpu_sc as plsc

_NC = 2
_NS = 16
_LANES = 16
_NW = _NC * _NS
_CHUNKS = 4


def _mesh():
    return plsc.VectorSubcoreMesh(core_axis_name="c", subcore_axis_name="s")


@functools.lru_cache(maxsize=None)
def _build_k1(batch: int, n_fields: int):
    s_per_w = batch // _NW
    s_chunk = s_per_w // _CHUNKS
    g_chunk = s_chunk // _LANES
    n_per_w = s_per_w * n_fields

    @functools.partial(
        pl.kernel,
        mesh=_mesh(),
        out_type=jax.ShapeDtypeStruct((batch * n_fields,), jnp.int32),
        compiler_params=pltpu.CompilerParams(needs_layout_passes=False),
        scratch_types=[
            pltpu.VMEM((s_per_w, n_fields), jnp.int32),
            pltpu.VMEM((n_per_w,), jnp.int32),
        ],
    )
    def k1(x_hbm, idx_hbm, x_v, idx_v):
        wid = lax.axis_index("s") * _NC + lax.axis_index("c")
        pltpu.sync_copy(x_hbm.at[pl.ds(wid * s_per_w, s_per_w), :], x_v)
        lane = lax.iota(jnp.int32, _LANES)

        def grp(t, carry):
            c = t // g_chunk
            g = t - c * g_chunk
            sl = c * s_chunk + g * _LANES + lane
            base = c * s_chunk * n_fields + g * _LANES
            for j in range(n_fields):
                v = plsc.load_gather(x_v, [sl, lane * 0 + j])
                idx_v[pl.ds(base + j * s_chunk, _LANES)] = v
            return carry

        lax.fori_loop(0, _CHUNKS * g_chunk, grp, 0)
        pltpu.sync_copy(idx_v, idx_hbm.at[pl.ds(wid * n_per_w, n_per_w)])

    return k1


@functools.lru_cache(maxsize=None)
def _build_k2(batch: int, n_fields: int):
    s_per_w = batch // _NW
    s_chunk = s_per_w // _CHUNKS
    g_chunk = s_chunk // _LANES
    n_per_w = s_per_w * n_fields
    n_chunk = s_chunk * n_fields

    @functools.partial(
        pl.kernel,
        mesh=_mesh(),
        out_type=jax.ShapeDtypeStruct((batch,), jnp.float32),
        compiler_params=pltpu.CompilerParams(needs_layout_passes=False),
        scratch_types=(
            [pltpu.VMEM((n_per_w,), jnp.int32)]
            + [pltpu.VMEM((n_chunk,), jnp.float32) for _ in range(_CHUNKS)]
            + [pltpu.VMEM((1,), jnp.float32),
               pltpu.VMEM((s_per_w,), jnp.float32)]
            + [pltpu.SemaphoreType.DMA for _ in range(_CHUNKS)]
        ),
    )
    def k2(idx_hbm, tab_hbm, bias_hbm, out_hbm, *scr):
        idx_v = scr[0]
        val_vs = scr[1:1 + _CHUNKS]
        bias_v, out_v = scr[1 + _CHUNKS:3 + _CHUNKS]
        sems = scr[3 + _CHUNKS:]
        wid = lax.axis_index("s") * _NC + lax.axis_index("c")
        pltpu.sync_copy(idx_hbm.at[pl.ds(wid * n_per_w, n_per_w)], idx_v)
        pltpu.sync_copy(bias_hbm, bias_v)
        bias_vec = plsc.load_gather(bias_v, [lax.iota(jnp.int32, _LANES) * 0])

        def start(c):
            return pltpu.async_copy(
                tab_hbm.at[idx_v.at[pl.ds(c * n_chunk, n_chunk)]],
                val_vs[c], sems[c])

        def reduce(c):
            val_v = val_vs[c]

            def grp(g, carry):
                acc = bias_vec
                for j in range(n_fields):
                    acc = acc + val_v[pl.ds(j * s_chunk + g * _LANES, _LANES)]
                sig = 1.0 / (1.0 + jnp.exp(-acc))
                out_v[pl.ds(c * s_chunk + g * _LANES, _LANES)] = sig
                return carry

            lax.fori_loop(0, g_chunk, grp, 0)

        dmas = [start(c) for c in range(_CHUNKS)]
        for c in range(_CHUNKS):
            dmas[c].wait()
            reduce(c)
        pltpu.sync_copy(out_v, out_hbm.at[pl.ds(wid * s_per_w, s_per_w)])

    return k2


def kernel(x, table, bias):
    batch, n_fields = x.shape
    tab_flat = table.astype(jnp.float32).reshape(-1)
    bias1 = bias.reshape(-1)[:1].astype(jnp.float32)
    idx_flat = _build_k1(batch, n_fields)(x.astype(jnp.int32))
    out = _build_k2(batch, n_fields)(idx_flat, tab_flat, bias1)
    return out.reshape(batch, 1)
